# Initial kernel scaffold; baseline (speedup 1.0000x reference)
import jax
import jax.numpy as jnp
from jax.experimental import pallas as pl


def kernel(flat, cu_seqlens):
    raise NotImplementedError("write your pallas kernel here")


# speedup vs baseline: 2.5064x; 2.5064x over previous
import functools

import jax
import jax.numpy as jnp
from jax import lax
from jax.experimental import pallas as pl
from jax.experimental.pallas import tpu as pltpu
from jax.experimental.pallas import tpu_sc as plsc

_B = 8
_M = 2048
_D = 1024
_TOTAL = 8192
_NW = 32
_VR = _TOTAL // _NW
_PR = (_B * _M - _TOTAL) // _NW
_C = 32
_NCV = _VR // _C
_NCP = _PR // _C
_MB = _B * _M // _NW


def _make_sc_kernel():
  mesh = plsc.VectorSubcoreMesh(core_axis_name="c", subcore_axis_name="s")

  @functools.partial(
      pl.kernel,
      mesh=mesh,
      out_type=(
          jax.ShapeDtypeStruct((_B * _M, _D), jnp.float32),
          jax.ShapeDtypeStruct((_B * _M,), jnp.int32),
      ),
      scratch_types=[
          pltpu.VMEM((128,), jnp.int32),
          pltpu.VMEM((_NCV + _NCP, _C), jnp.int32),
          pltpu.VMEM((2, _C, _D), jnp.float32),
          pltpu.VMEM((_C, _D), jnp.float32),
          pltpu.VMEM((_MB,), jnp.int32),
          pltpu.SemaphoreType.DMA,
          pltpu.SemaphoreType.DMA,
          pltpu.SemaphoreType.DMA,
      ],
  )
  def k(flat_hbm, cu_hbm, z_hbm, out_hbm, mask_hbm,
        cu_v, idx_v, bufs, zbuf, mbuf, gsem, ssem, psem):
    wid = lax.axis_index("s") * 2 + lax.axis_index("c")
    pltpu.sync_copy(cu_hbm, cu_v)
    pltpu.sync_copy(z_hbm, zbuf)

    iota = lax.iota(jnp.int32, 16)
    cu_vec = cu_v[pl.ds(0, 16)]
    cus = [cu_vec[i] for i in range(_B + 1)]
    one = jnp.int32(1)
    zero = jnp.int32(0)

    vbase = wid * _VR
    for c in range(_NCV):
      for h in range(2):
        iv = vbase + c * _C + h * 16 + iota
        bm = jnp.where(iv >= cus[1], jnp.int32(_M), zero)
        cu_b = jnp.where(iv >= cus[1], cus[1] - cus[0], zero)
        for t in range(2, _B):
          bm = bm + jnp.where(iv >= cus[t], jnp.int32(_M), zero)
          cu_b = cu_b + jnp.where(iv >= cus[t], cus[t] - cus[t - 1], zero)
        idx_v[c, pl.ds(h * 16, 16)] = bm + iv - cu_b

    pbase = wid * _PR
    for c in range(_NCP):
      for h in range(2):
        jv = pbase + c * _C + h * 16 + iota
        cu_b1 = jnp.zeros((16,), jnp.int32) + cus[1]
        for t in range(1, _B):
          cu_b1 = cu_b1 + jnp.where(
              jv >= t * _M - cus[t], cus[t + 1] - cus[t], zero)
        idx_v[_NCV + c, pl.ds(h * 16, 16)] = cu_b1 + jv

    for c in range(_NCP):
      pltpu.async_copy(zbuf, out_hbm.at[idx_v.at[_NCV + c]], psem)

    def gather(c):
      return pltpu.async_copy(
          flat_hbm.at[pl.ds(vbase + c * _C, _C)], bufs.at[c % 2], gsem)

    gather(0)
    for c in range(_NCV):
      pltpu.make_async_copy(
          flat_hbm.at[pl.ds(vbase + c * _C, _C)], bufs.at[c % 2], gsem).wait()
      pltpu.async_copy(bufs.at[c % 2], out_hbm.at[idx_v.at[c]], ssem)
      if c >= 1:
        pltpu.make_async_copy(
            bufs.at[(c - 1) % 2], out_hbm.at[idx_v.at[c - 1]], ssem).wait()
      if c + 1 < _NCV:
        gather(c + 1)

    bvec = jnp.zeros((16,), jnp.int32) + wid // 4
    lenb = jnp.where(bvec == 0, cus[1] - cus[0], zero)
    for t in range(1, _B):
      lenb = lenb + jnp.where(bvec == t, cus[t + 1] - cus[t], zero)
    m0 = (wid % 4) * _MB

    def mask_body(g, carry):
      mv = m0 + g * 16 + iota
      mbuf[pl.ds(g * 16, 16)] = jnp.where(mv >= lenb, one, zero)
      return carry

    lax.fori_loop(0, _MB // 16, mask_body, zero)
    pltpu.sync_copy(mbuf, mask_hbm.at[pl.ds(wid * _MB, _MB)])

    pltpu.make_async_copy(
        bufs.at[(_NCV - 1) % 2], out_hbm.at[idx_v.at[_NCV - 1]], ssem).wait()
    for c in range(_NCP):
      pltpu.make_async_copy(zbuf, out_hbm.at[idx_v.at[_NCV + c]], psem).wait()

  return k


_sc_pad = _make_sc_kernel()


@jax.jit
def kernel(flat, cu_seqlens):
  zeros = jnp.zeros((_C, _D), jnp.float32)
  cu16 = jnp.pad(cu_seqlens.astype(jnp.int32), (0, 128 - (_B + 1)))
  out_flat, mask_i = _sc_pad(flat, cu16, zeros)
  padded = out_flat.reshape(_B, _M, _D)
  mask = mask_i.reshape(_B, _M).astype(jnp.bool_)
  return padded, mask
